# speedup vs baseline: 1.1135x; 1.1135x over previous
import jax
import jax.numpy as jnp
from jax import lax
from jax.experimental import pallas as pl
from jax.experimental.pallas import tpu as pltpu
from jax.experimental.pallas import tpu_sc as plsc

_B, _S, _D = 4, 2048, 1024
_N = _B * _S
_SCALE = 32.0

_NC, _NS, _L = 2, 16, 16
_NW = _NC * _NS
_TPW = _N // _NW
_T = 32
_NCHUNK = _TPW // _T


def _sc_body(ids_hbm, pids_hbm, wtab_hbm, ptab_hbm, out_hbm,
             idx_v, pidx_v, wbuf, pbuf, sem):
    wid = lax.axis_index("s") * _NC + lax.axis_index("c")

    def chunk(c, carry):
        base = wid * _TPW + c * _T
        pltpu.sync_copy(ids_hbm.at[pl.ds(base, _T)], idx_v)
        pltpu.sync_copy(pids_hbm.at[pl.ds(base, _T)], pidx_v)
        cp_w = pltpu.async_copy(wtab_hbm.at[idx_v], wbuf, sem)
        cp_p = pltpu.async_copy(ptab_hbm.at[pidx_v], pbuf, sem)
        cp_w.wait()
        cp_p.wait()

        def row(r, rc):
            for j in range(_D // _L):
                s = pl.ds(j * _L, _L)
                wbuf[r, s] = wbuf[r, s] * _SCALE + pbuf[r, s]
            return rc

        lax.fori_loop(0, _T, row, 0)
        pltpu.sync_copy(wbuf, out_hbm.at[pl.ds(base, _T)])
        return carry

    lax.fori_loop(0, _NCHUNK, chunk, 0)


def kernel(input_ids, position_ids, word_table, pos_table):
    ids = input_ids.reshape(_N).astype(jnp.int32)
    pids = position_ids.reshape(_N).astype(jnp.int32)
    mesh = plsc.VectorSubcoreMesh(
        core_axis_name="c", subcore_axis_name="s",
        num_cores=_NC, num_subcores=_NS)
    out = pl.kernel(
        _sc_body,
        out_type=jax.ShapeDtypeStruct((_N, _D), jnp.float32),
        mesh=mesh,
        scratch_types=[
            pltpu.VMEM((_T,), jnp.int32),
            pltpu.VMEM((_T,), jnp.int32),
            pltpu.VMEM((_T, _D), jnp.float32),
            pltpu.VMEM((_T, _D), jnp.float32),
            pltpu.SemaphoreType.DMA,
        ],
    )(ids, pids, word_table, pos_table)
    return out.reshape(_B, _S, _D)


# speedup vs baseline: 1.6549x; 1.4863x over previous
import jax
import jax.numpy as jnp
from jax import lax
from jax.experimental import pallas as pl
from jax.experimental.pallas import tpu as pltpu
from jax.experimental.pallas import tpu_sc as plsc

_B, _S, _D = 4, 2048, 1024
_N = _B * _S
_SCALE = 32.0

_NC, _NS, _L = 2, 16, 16
_NW = _NC * _NS
_TPW = _N // _NW
_T = 16
_NCHUNK = _TPW // _T
_NSLOT = 2
_NGRP = _NCHUNK // _NSLOT


def _sc_body(ids_hbm, pids_hbm, wtab_hbm, ptab_hbm, out_hbm,
             idx_v, pidx_v, wbuf0, wbuf1, pbuf0, pbuf1,
             sg0, sg1, sw0, sw1):
    wbufs = (wbuf0, wbuf1)
    pbufs = (pbuf0, pbuf1)
    sgs = (sg0, sg1)
    sws = (sw0, sw1)
    wid = lax.axis_index("s") * _NC + lax.axis_index("c")
    tok0 = wid * _TPW

    pltpu.sync_copy(ids_hbm.at[pl.ds(tok0, _TPW)], idx_v)
    pltpu.sync_copy(pids_hbm.at[pl.ds(tok0, _TPW)], pidx_v)

    def issue_gathers(c, b):
        off = c * _T
        pltpu.async_copy(wtab_hbm.at[idx_v.at[pl.ds(off, _T)]], wbufs[b], sgs[b])
        pltpu.async_copy(ptab_hbm.at[pidx_v.at[pl.ds(off, _T)]], pbufs[b], sgs[b])

    def wait_gathers(c, b):
        off = c * _T
        pltpu.make_async_copy(
            wtab_hbm.at[idx_v.at[pl.ds(off, _T)]], wbufs[b], sgs[b]).wait()
        pltpu.make_async_copy(
            ptab_hbm.at[pidx_v.at[pl.ds(off, _T)]], pbufs[b], sgs[b]).wait()

    for b in range(_NSLOT):
        issue_gathers(b, b)

    def group(g, carry):
        for b in range(_NSLOT):
            c = g * _NSLOT + b
            wait_gathers(c, b)

            def row(r, rc):
                for j in range(_D // _L):
                    s = pl.ds(j * _L, _L)
                    wbufs[b][r, s] = wbufs[b][r, s] * _SCALE + pbufs[b][r, s]
                return rc

            lax.fori_loop(0, _T, row, 0)

            @pl.when(g > 0)
            def _():
                pltpu.make_async_copy(
                    wbufs[b], out_hbm.at[pl.ds(tok0, _T)], sws[b]).wait()

            pltpu.async_copy(
                wbufs[b], out_hbm.at[pl.ds(tok0 + c * _T, _T)], sws[b])

            @pl.when(g < _NGRP - 1)
            def _():
                issue_gathers(c + _NSLOT, b)
        return carry

    lax.fori_loop(0, _NGRP, group, 0)

    for b in range(_NSLOT):
        pltpu.make_async_copy(
            wbufs[b], out_hbm.at[pl.ds(tok0, _T)], sws[b]).wait()


def kernel(input_ids, position_ids, word_table, pos_table):
    ids = input_ids.reshape(_N).astype(jnp.int32)
    pids = position_ids.reshape(_N).astype(jnp.int32)
    mesh = plsc.VectorSubcoreMesh(
        core_axis_name="c", subcore_axis_name="s",
        num_cores=_NC, num_subcores=_NS)
    out = pl.kernel(
        _sc_body,
        out_type=jax.ShapeDtypeStruct((_N, _D), jnp.float32),
        mesh=mesh,
        scratch_types=[
            pltpu.VMEM((_TPW,), jnp.int32),
            pltpu.VMEM((_TPW,), jnp.int32),
            pltpu.VMEM((_T, _D), jnp.float32),
            pltpu.VMEM((_T, _D), jnp.float32),
            pltpu.VMEM((_T, _D), jnp.float32),
            pltpu.VMEM((_T, _D), jnp.float32),
            pltpu.SemaphoreType.DMA,
            pltpu.SemaphoreType.DMA,
            pltpu.SemaphoreType.DMA,
            pltpu.SemaphoreType.DMA,
        ],
    )(ids, pids, word_table, pos_table)
    return out.reshape(_B, _S, _D)


# speedup vs baseline: 1.6554x; 1.0003x over previous
import jax
import jax.numpy as jnp
from jax import lax
from jax.experimental import pallas as pl
from jax.experimental.pallas import tpu as pltpu
from jax.experimental.pallas import tpu_sc as plsc

_B, _S, _D = 4, 2048, 1024
_N = _B * _S
_SCALE = 32.0

_NC, _NS, _L = 2, 16, 16
_NW = _NC * _NS
_TPW = _N // _NW
_T = 16
_NCHUNK = _TPW // _T
_NSLOT = 2
_NGRP = _NCHUNK // _NSLOT


def _sc_body(ids_hbm, pids_hbm, wtab_hbm, ptab_hbm, out_hbm,
             idx_v, pidx_v, wbuf0, wbuf1, pbuf0, pbuf1, obuf0, obuf1,
             sg0, sg1, sw0, sw1):
    wbufs = (wbuf0, wbuf1)
    pbufs = (pbuf0, pbuf1)
    obufs = (obuf0, obuf1)
    sgs = (sg0, sg1)
    sws = (sw0, sw1)
    wid = lax.axis_index("s") * _NC + lax.axis_index("c")
    tok0 = wid * _TPW

    pltpu.sync_copy(ids_hbm.at[pl.ds(tok0, _TPW)], idx_v)
    pltpu.sync_copy(pids_hbm.at[pl.ds(tok0, _TPW)], pidx_v)

    def issue_gathers(c, b):
        off = c * _T
        pltpu.async_copy(wtab_hbm.at[idx_v.at[pl.ds(off, _T)]], wbufs[b], sgs[b])
        pltpu.async_copy(ptab_hbm.at[pidx_v.at[pl.ds(off, _T)]], pbufs[b], sgs[b])

    def wait_gathers(c, b):
        off = c * _T
        pltpu.make_async_copy(
            wtab_hbm.at[idx_v.at[pl.ds(off, _T)]], wbufs[b], sgs[b]).wait()
        pltpu.make_async_copy(
            ptab_hbm.at[pidx_v.at[pl.ds(off, _T)]], pbufs[b], sgs[b]).wait()

    def wait_wb(b):
        pltpu.make_async_copy(
            obufs[b], out_hbm.at[pl.ds(tok0, _T)], sws[b]).wait()

    for b in range(_NSLOT):
        issue_gathers(b, b)

    def group(g, carry):
        for b in range(_NSLOT):
            c = g * _NSLOT + b

            @pl.when(g > 0)
            def _():
                wait_wb(b)

            wait_gathers(c, b)

            def row(r, rc):
                for j in range(_D // _L):
                    s = pl.ds(j * _L, _L)
                    obufs[b][r, s] = wbufs[b][r, s] * _SCALE + pbufs[b][r, s]
                return rc

            lax.fori_loop(0, _T, row, 0)

            pltpu.async_copy(
                obufs[b], out_hbm.at[pl.ds(tok0 + c * _T, _T)], sws[b])

            @pl.when(g < _NGRP - 1)
            def _():
                issue_gathers(c + _NSLOT, b)
        return carry

    lax.fori_loop(0, _NGRP, group, 0)

    for b in range(_NSLOT):
        wait_wb(b)


def kernel(input_ids, position_ids, word_table, pos_table):
    ids = input_ids.reshape(_N).astype(jnp.int32)
    pids = position_ids.reshape(_N).astype(jnp.int32)
    mesh = plsc.VectorSubcoreMesh(
        core_axis_name="c", subcore_axis_name="s",
        num_cores=_NC, num_subcores=_NS)
    out = pl.kernel(
        _sc_body,
        out_type=jax.ShapeDtypeStruct((_N, _D), jnp.float32),
        mesh=mesh,
        scratch_types=[
            pltpu.VMEM((_TPW,), jnp.int32),
            pltpu.VMEM((_TPW,), jnp.int32),
            pltpu.VMEM((_T, _D), jnp.float32),
            pltpu.VMEM((_T, _D), jnp.float32),
            pltpu.VMEM((_T, _D), jnp.float32),
            pltpu.VMEM((_T, _D), jnp.float32),
            pltpu.VMEM((_T, _D), jnp.float32),
            pltpu.VMEM((_T, _D), jnp.float32),
            pltpu.SemaphoreType.DMA,
            pltpu.SemaphoreType.DMA,
            pltpu.SemaphoreType.DMA,
            pltpu.SemaphoreType.DMA,
        ],
    )(ids, pids, word_table, pos_table)
    return out.reshape(_B, _S, _D)


# speedup vs baseline: 1.6668x; 1.0069x over previous
import jax
import jax.numpy as jnp
from jax import lax
from jax.experimental import pallas as pl
from jax.experimental.pallas import tpu as pltpu
from jax.experimental.pallas import tpu_sc as plsc

_B, _S, _D = 4, 2048, 1024
_N = _B * _S
_SCALE = 32.0

_NC, _NS, _L = 2, 16, 16
_NW = _NC * _NS
_TPW = _N // _NW
_WPR = _S // _TPW
_T = 16
_NCHUNK = _TPW // _T
_NSLOT = 2
_NGRP = _NCHUNK // _NSLOT


def _sc_body(ids_hbm, pids_hbm, wtab_hbm, ptab_hbm, out_hbm,
             idx_v, pidx_v, wbuf0, wbuf1, pbuf0, pbuf1, obuf0, obuf1,
             sg0, sg1, sw0, sw1):
    wbufs = (wbuf0, wbuf1)
    pbufs = (pbuf0, pbuf1)
    obufs = (obuf0, obuf1)
    sgs = (sg0, sg1)
    sws = (sw0, sw1)
    wid = lax.axis_index("s") * _NC + lax.axis_index("c")
    tok0 = wid * _TPW
    row = wid // _WPR
    col0 = (wid % _WPR) * _TPW

    pltpu.async_copy(ids_hbm.at[row, pl.ds(col0, _TPW)], idx_v, sg0)
    pltpu.async_copy(pids_hbm.at[row, pl.ds(col0, _TPW)], pidx_v, sg1)
    pltpu.make_async_copy(ids_hbm.at[row, pl.ds(col0, _TPW)], idx_v, sg0).wait()
    pltpu.make_async_copy(pids_hbm.at[row, pl.ds(col0, _TPW)], pidx_v, sg1).wait()

    def issue_gathers(c, b):
        off = c * _T
        pltpu.async_copy(wtab_hbm.at[idx_v.at[pl.ds(off, _T)]], wbufs[b], sgs[b])
        pltpu.async_copy(ptab_hbm.at[pidx_v.at[pl.ds(off, _T)]], pbufs[b], sgs[b])

    def wait_gathers(c, b):
        off = c * _T
        pltpu.make_async_copy(
            wtab_hbm.at[idx_v.at[pl.ds(off, _T)]], wbufs[b], sgs[b]).wait()
        pltpu.make_async_copy(
            ptab_hbm.at[pidx_v.at[pl.ds(off, _T)]], pbufs[b], sgs[b]).wait()

    def wait_wb(b):
        pltpu.make_async_copy(
            obufs[b], out_hbm.at[pl.ds(tok0, _T)], sws[b]).wait()

    for b in range(_NSLOT):
        issue_gathers(b, b)

    def group(g, carry):
        for b in range(_NSLOT):
            c = g * _NSLOT + b

            @pl.when(g > 0)
            def _():
                wait_wb(b)

            wait_gathers(c, b)

            def row_fn(r, rc):
                for j in range(_D // _L):
                    s = pl.ds(j * _L, _L)
                    obufs[b][r, s] = wbufs[b][r, s] * _SCALE + pbufs[b][r, s]
                return rc

            lax.fori_loop(0, _T, row_fn, 0)

            pltpu.async_copy(
                obufs[b], out_hbm.at[pl.ds(tok0 + c * _T, _T)], sws[b])

            @pl.when(g < _NGRP - 1)
            def _():
                issue_gathers(c + _NSLOT, b)
        return carry

    lax.fori_loop(0, _NGRP, group, 0)

    for b in range(_NSLOT):
        wait_wb(b)


def kernel(input_ids, position_ids, word_table, pos_table):
    ids = input_ids.astype(jnp.int32)
    pids = position_ids.astype(jnp.int32)
    mesh = plsc.VectorSubcoreMesh(
        core_axis_name="c", subcore_axis_name="s",
        num_cores=_NC, num_subcores=_NS)
    out = pl.kernel(
        _sc_body,
        out_type=jax.ShapeDtypeStruct((_N, _D), jnp.float32),
        mesh=mesh,
        scratch_types=[
            pltpu.VMEM((_TPW,), jnp.int32),
            pltpu.VMEM((_TPW,), jnp.int32),
            pltpu.VMEM((_T, _D), jnp.float32),
            pltpu.VMEM((_T, _D), jnp.float32),
            pltpu.VMEM((_T, _D), jnp.float32),
            pltpu.VMEM((_T, _D), jnp.float32),
            pltpu.VMEM((_T, _D), jnp.float32),
            pltpu.VMEM((_T, _D), jnp.float32),
            pltpu.SemaphoreType.DMA,
            pltpu.SemaphoreType.DMA,
            pltpu.SemaphoreType.DMA,
            pltpu.SemaphoreType.DMA,
        ],
    )(ids, pids, word_table, pos_table)
    return out.reshape(_B, _S, _D)
